# baseline (speedup 1.0000x reference)
import functools

import jax
import jax.numpy as jnp
from jax import lax
from jax.experimental import pallas as pl
from jax.experimental.pallas import tpu as pltpu
from jax.experimental.pallas import tpu_sc as plsc

N = 10000
NPAD = 10240
DH = 512
E_RAW = 160000
E_FULL = E_RAW + N
KR1 = 86
NT1 = KR1 * 128
EPAD = NT1 * 16
NRANGE = 128
RNODES = NPAD // NRANGE
RPT = NRANGE // 32
_F32 = jnp.float32
_I32 = jnp.int32


def _mm_call(xp, w, a128, relu_in):
    k = w.shape[0]
    nblk = NPAD // 512

    def body(x_ref, w_ref, a_ref, h_ref, av_ref):
        xb = x_ref[...]
        if relu_in:
            xb = jnp.maximum(xb, 0.0)
        h = jnp.dot(xb, w_ref[...], preferred_element_type=_F32)
        h_ref[...] = h
        av_ref[...] = jnp.dot(h, a_ref[...], preferred_element_type=_F32)

    return pl.pallas_call(
        body,
        grid=(nblk,),
        in_specs=[
            pl.BlockSpec((512, k), lambda i: (i, 0)),
            pl.BlockSpec((k, DH), lambda i: (0, 0)),
            pl.BlockSpec((DH, 128), lambda i: (0, 0)),
        ],
        out_specs=[
            pl.BlockSpec((512, DH), lambda i: (i, 0)),
            pl.BlockSpec((512, 128), lambda i: (i, 0)),
        ],
        out_shape=[
            jax.ShapeDtypeStruct((NPAD, DH), _F32),
            jax.ShapeDtypeStruct((NPAD, 128), _F32),
        ],
    )(xp, w, a128)


def _edge_body(src3_hbm, dst3_hbm, srcf_hbm, dstf_hbm, cuts_hbm,
               as_hbm, ad_hbm, h_hbm, b_hbm, out_hbm,
               src_v, dst_v, as_v, ad_v, dn_v, cuts_v, b_v, est_v,
               acc_v, rows_v, idx_v, lidx_v, w_v, denomS,
               dsem0, dsem1, gsem0, gsem1):
    c = lax.axis_index("c")
    s = lax.axis_index("s")
    dsems = (dsem0, dsem1)
    gsems = (gsem0, gsem1)

    pltpu.sync_copy(src3_hbm.at[s], src_v)
    pltpu.sync_copy(dst3_hbm.at[s], dst_v)
    pltpu.sync_copy(as_hbm, as_v)
    pltpu.sync_copy(ad_hbm, ad_v)
    pltpu.sync_copy(cuts_hbm, cuts_v)
    pltpu.sync_copy(b_hbm, b_v)

    for q in range(8):
        est_v[0, pl.ds(q * 16, 16)] = jnp.zeros((16,), _F32)
    for q in range(5):
        pltpu.sync_copy(est_v.at[0], denomS.at[pl.ds(s * 640 + q * 128, 128)])
    plsc.subcore_barrier()

    def eexp_row(kk, b):
        def g_body(g, _):
            sl = pl.ds(g * 16, 16)
            s16 = src_v[kk, sl]
            d16 = dst_v[kk, sl]
            av = plsc.load_gather(as_v, [s16])
            dv = plsc.load_gather(ad_v, [d16])
            e = av + dv
            e = jnp.where(e >= 0.0, e, 0.2 * e)
            est_v[b, sl] = jnp.exp(e)
            return 0
        lax.fori_loop(0, 8, g_body, 0)

    def p1_pair(r2, _):
        for b in range(2):
            r = r2 * 2 + b

            @pl.when(r >= 2)
            def _():
                pltpu.make_async_copy(est_v.at[b], denomS.at[dst_v.at[r - 2]],
                                      dsems[b]).wait()
            eexp_row(r, b)
            pltpu.async_copy(est_v.at[b], denomS.at[dst_v.at[r]], dsems[b],
                             add=True)
        return 0
    lax.fori_loop(0, KR1 // 2, p1_pair, 0)
    pltpu.make_async_copy(est_v.at[0], denomS.at[dst_v.at[KR1 - 2]],
                          dsems[0]).wait()
    pltpu.make_async_copy(est_v.at[1], denomS.at[dst_v.at[KR1 - 1]],
                          dsems[1]).wait()
    plsc.subcore_barrier()

    pltpu.sync_copy(denomS, dn_v)

    w_id = c * 16 + s
    lanes = lax.iota(_I32, 16)

    def scalar_at(r):
        v = plsc.load_gather(cuts_v, [jnp.broadcast_to(r, (16,)).astype(_I32)])
        return jnp.max(v)

    for rp in range(RPT):
        rid = w_id * RPT + rp
        lo = rid * RNODES
        hi = lo + RNODES
        e_start = scalar_at(rid)
        e_end = scalar_at(rid + 1)
        e0 = jnp.bitwise_and(e_start, jnp.int32(-16))
        nseg = (e_end - e0 + 127) // 128

        def zacc(j, _):
            for k2 in range(DH // 16):
                acc_v[j, pl.ds(k2 * 16, 16)] = jnp.zeros((16,), _F32)
            return 0
        lax.fori_loop(0, RNODES, zacc, 0)

        def prep(g, bb):
            sl = pl.ds(g * 16, 16)
            s16 = src_v[0, sl]
            d16 = dst_v[0, sl]
            av = plsc.load_gather(as_v, [s16])
            dv = plsc.load_gather(ad_v, [d16])
            e = av + dv
            e = jnp.where(e >= 0.0, e, 0.2 * e)
            dn = plsc.load_gather(dn_v, [d16])
            alpha = jnp.exp(e) / (dn + 1e-16)
            inr = (d16 >= lo) & (d16 < hi)
            idx_v[bb, :] = s16
            lidx_v[bb, :] = jnp.where(inr, d16 - lo, 0)
            w_v[bb, :] = jnp.where(inr, alpha, 0.0)

        def gstart(bb):
            pltpu.async_copy(h_hbm.at[idx_v.at[bb]], rows_v.at[bb], gsems[bb])

        def gwait(bb):
            pltpu.make_async_copy(h_hbm.at[idx_v.at[bb]], rows_v.at[bb],
                                  gsems[bb]).wait()

        def seg_body(si, _):
            eb = pl.multiple_of(e0 + si * 128, 16)
            pltpu.sync_copy(srcf_hbm.at[pl.ds(eb, 128)], src_v.at[0])
            pltpu.sync_copy(dstf_hbm.at[pl.ds(eb, 128)], dst_v.at[0])
            prep(0, 0)
            gstart(0)
            for g in range(8):
                b = g % 2
                gwait(b)
                if g < 7:
                    prep(g + 1, 1 - b)
                    gstart(1 - b)
                w16 = w_v[b, :]
                l16 = lidx_v[b, :]
                bb16 = jnp.full((16,), b, _I32)

                def col_body(c8, _):
                    base = c8 * 4
                    for u in range(4):
                        cs = jnp.broadcast_to(base + u, (16,)).astype(_I32)
                        vals = plsc.load_gather(rows_v, [bb16, lanes, cs])
                        plsc.addupdate_scatter(acc_v, [l16, cs], vals * w16)
                    return 0
                lax.fori_loop(0, DH // 4, col_body, 0)
            return 0
        lax.fori_loop(0, nseg, seg_body, 0)

        def bias_row(j, _):
            for k2 in range(DH // 16):
                sl = pl.ds(k2 * 16, 16)
                acc_v[j, sl] = acc_v[j, sl] + b_v[sl]
            return 0
        lax.fori_loop(0, RNODES, bias_row, 0)
        pltpu.sync_copy(acc_v, out_hbm.at[pl.ds(lo, RNODES)])


@functools.cache
def _edge_call():
  return pl.kernel(
    _edge_body,
    out_type=jax.ShapeDtypeStruct((NPAD, DH), _F32),
    mesh=plsc.VectorSubcoreMesh(core_axis_name="c", subcore_axis_name="s",
                                num_cores=2, num_subcores=16),
    compiler_params=pltpu.CompilerParams(needs_layout_passes=False),
    scratch_types=[
        pltpu.VMEM((KR1, 128), _I32),
        pltpu.VMEM((KR1, 128), _I32),
        pltpu.VMEM((NPAD,), _F32),
        pltpu.VMEM((NPAD,), _F32),
        pltpu.VMEM((NPAD,), _F32),
        pltpu.VMEM((144,), _I32),
        pltpu.VMEM((DH,), _F32),
        pltpu.VMEM((2, 128), _F32),
        pltpu.VMEM((RNODES, DH), _F32),
        pltpu.VMEM((2, 16, DH), _F32),
        pltpu.VMEM((2, 16), _I32),
        pltpu.VMEM((2, 16), _I32),
        pltpu.VMEM((2, 16), _F32),
        pltpu.VMEM_SHARED((NPAD,), _F32),
        pltpu.SemaphoreType.DMA,
        pltpu.SemaphoreType.DMA,
        pltpu.SemaphoreType.DMA,
        pltpu.SemaphoreType.DMA,
    ],
  )


def _edge_layer(src3, dst3, srcf, dstf, cuts, as_, ad_, h, b):
    return _edge_call()(src3, dst3, srcf, dstf, cuts, as_, ad_, h, b)


@jax.jit
def kernel(x, edge_index, W1, a_src1, a_dst1, b1, W2, a_src2, a_dst2, b2):
    npad_e = EPAD - E_FULL
    loop = jnp.arange(N, dtype=_I32)
    src = jnp.concatenate([edge_index[0].astype(_I32), loop,
                           jnp.zeros((npad_e,), _I32)])
    dst = jnp.concatenate([edge_index[1].astype(_I32), loop,
                           N + (jnp.arange(npad_e, dtype=_I32) % (NPAD - N))])
    order = jnp.argsort(dst)
    srcf = src[order]
    dstf = dst[order]
    src3 = srcf.reshape(16, KR1, 128)
    dst3 = dstf.reshape(16, KR1, 128)
    cuts = jnp.searchsorted(dstf, jnp.arange(NRANGE, dtype=_I32) * RNODES,
                            side="left").astype(_I32)
    cuts = jnp.concatenate([cuts, jnp.full((144 - NRANGE,), EPAD, _I32)])
    srcf = jnp.concatenate([srcf, jnp.zeros((128,), _I32)])
    dstf = jnp.concatenate([dstf, jnp.full((128,), NPAD - 1, _I32)])

    xp = jnp.pad(x, ((0, NPAD - N), (0, 0)))
    a128_1 = jnp.zeros((DH, 128), _F32).at[:, 0].set(a_src1).at[:, 1].set(a_dst1)
    a128_2 = jnp.zeros((DH, 128), _F32).at[:, 0].set(a_src2).at[:, 1].set(a_dst2)

    h1, av1 = _mm_call(xp, W1, a128_1, relu_in=False)
    out1 = _edge_layer(src3, dst3, srcf, dstf, cuts,
                       av1[:, 0], av1[:, 1], h1, b1)
    h2, av2 = _mm_call(out1, W2, a128_2, relu_in=True)
    out2 = _edge_layer(src3, dst3, srcf, dstf, cuts,
                       av2[:, 0], av2[:, 1], h2, b2)
    return out2[:N]


# speedup vs baseline: 3.5483x; 3.5483x over previous
import functools

import jax
import jax.numpy as jnp
from jax import lax
from jax.experimental import pallas as pl
from jax.experimental.pallas import tpu as pltpu
from jax.experimental.pallas import tpu_sc as plsc

N = 10000
NPAD = 10240
DH = 512
E_RAW = 160000
E_FULL = E_RAW + N
KR1 = 86
NT1 = KR1 * 128
EPAD = NT1 * 16
NRANGE = 128
RNODES = NPAD // NRANGE
RPT = NRANGE // 32
_F32 = jnp.float32
_I32 = jnp.int32


def _mm_call(xp, w, a128, relu_in):
    k = w.shape[0]
    nblk = NPAD // 512

    def body(x_ref, w_ref, a_ref, h_ref, av_ref):
        xb = x_ref[...]
        if relu_in:
            xb = jnp.maximum(xb, 0.0)
        h = jnp.dot(xb, w_ref[...], preferred_element_type=_F32)
        h_ref[...] = h
        av_ref[...] = jnp.dot(h, a_ref[...], preferred_element_type=_F32)

    return pl.pallas_call(
        body,
        grid=(nblk,),
        in_specs=[
            pl.BlockSpec((512, k), lambda i: (i, 0)),
            pl.BlockSpec((k, DH), lambda i: (0, 0)),
            pl.BlockSpec((DH, 128), lambda i: (0, 0)),
        ],
        out_specs=[
            pl.BlockSpec((512, DH), lambda i: (i, 0)),
            pl.BlockSpec((512, 128), lambda i: (i, 0)),
        ],
        out_shape=[
            jax.ShapeDtypeStruct((NPAD, DH), _F32),
            jax.ShapeDtypeStruct((NPAD, 128), _F32),
        ],
    )(xp, w, a128)


def _edge_body(src3_hbm, dst3_hbm, srcf_hbm, dstf_hbm, cuts_hbm,
               as_hbm, ad_hbm, h_hbm, b_hbm, out_hbm,
               src_v, dst_v, as_v, ad_v, dn_v, cuts_v, b_v, est_v,
               acc_v, rows_v, idx_v, lidx_v, w_v, denomS,
               dsem0, dsem1, gsem0, gsem1):
    c = lax.axis_index("c")
    s = lax.axis_index("s")
    dsems = (dsem0, dsem1)
    gsems = (gsem0, gsem1)

    pltpu.sync_copy(src3_hbm.at[s], src_v)
    pltpu.sync_copy(dst3_hbm.at[s], dst_v)
    pltpu.sync_copy(as_hbm, as_v)
    pltpu.sync_copy(ad_hbm, ad_v)
    pltpu.sync_copy(cuts_hbm, cuts_v)
    pltpu.sync_copy(b_hbm, b_v)

    for q in range(8):
        est_v[0, pl.ds(q * 16, 16)] = jnp.zeros((16,), _F32)
    for q in range(5):
        pltpu.sync_copy(est_v.at[0], denomS.at[pl.ds(s * 640 + q * 128, 128)])
    plsc.subcore_barrier()

    def eexp_row(kk, b):
        def g_body(g, _):
            sl = pl.ds(g * 16, 16)
            s16 = src_v[kk, sl]
            d16 = dst_v[kk, sl]
            av = plsc.load_gather(as_v, [s16])
            dv = plsc.load_gather(ad_v, [d16])
            e = av + dv
            e = jnp.where(e >= 0.0, e, 0.2 * e)
            est_v[b, sl] = jnp.exp(e)
            return 0
        lax.fori_loop(0, 8, g_body, 0)

    def p1_pair(r2, _):
        for b in range(2):
            r = r2 * 2 + b

            @pl.when(r >= 2)
            def _():
                pltpu.make_async_copy(est_v.at[b], denomS.at[dst_v.at[r - 2]],
                                      dsems[b]).wait()
            eexp_row(r, b)
            pltpu.async_copy(est_v.at[b], denomS.at[dst_v.at[r]], dsems[b],
                             add=True)
        return 0
    lax.fori_loop(0, KR1 // 2, p1_pair, 0)
    pltpu.make_async_copy(est_v.at[0], denomS.at[dst_v.at[KR1 - 2]],
                          dsems[0]).wait()
    pltpu.make_async_copy(est_v.at[1], denomS.at[dst_v.at[KR1 - 1]],
                          dsems[1]).wait()
    plsc.subcore_barrier()

    pltpu.sync_copy(denomS, dn_v)

    w_id = c * 16 + s
    lanes = lax.iota(_I32, 16)

    def scalar_at(r):
        v = plsc.load_gather(cuts_v, [jnp.broadcast_to(r, (16,)).astype(_I32)])
        return jnp.max(v)

    def range_body(rp, _):
        rid = w_id * RPT + rp
        lo = pl.multiple_of(rid * RNODES, RNODES)
        hi = lo + RNODES
        e_start = scalar_at(rid)
        e_end = scalar_at(rid + 1)
        e0 = jnp.bitwise_and(e_start, jnp.int32(-16))
        nseg = (e_end - e0 + 127) // 128

        def zacc(j, _):
            for k2 in range(DH // 16):
                acc_v[j, pl.ds(k2 * 16, 16)] = jnp.zeros((16,), _F32)
            return 0
        lax.fori_loop(0, RNODES, zacc, 0)

        def prep(g, bb):
            sl = pl.ds(g * 16, 16)
            s16 = src_v[0, sl]
            d16 = dst_v[0, sl]
            av = plsc.load_gather(as_v, [s16])
            dv = plsc.load_gather(ad_v, [d16])
            e = av + dv
            e = jnp.where(e >= 0.0, e, 0.2 * e)
            dn = plsc.load_gather(dn_v, [d16])
            alpha = jnp.exp(e) / (dn + 1e-16)
            inr = (d16 >= lo) & (d16 < hi)
            idx_v[bb, :] = s16
            lidx_v[bb, :] = jnp.where(inr, d16 - lo, 0)
            w_v[bb, :] = jnp.where(inr, alpha, 0.0)

        def gstart(bb):
            pltpu.async_copy(h_hbm.at[idx_v.at[bb]], rows_v.at[bb], gsems[bb])

        def gwait(bb):
            pltpu.make_async_copy(h_hbm.at[idx_v.at[bb]], rows_v.at[bb],
                                  gsems[bb]).wait()

        def seg_body(si, _):
            eb = pl.multiple_of(e0 + si * 128, 16)
            pltpu.sync_copy(srcf_hbm.at[pl.ds(eb, 128)], src_v.at[0])
            pltpu.sync_copy(dstf_hbm.at[pl.ds(eb, 128)], dst_v.at[0])
            prep(0, 0)
            gstart(0)
            for g in range(8):
                b = g % 2
                gwait(b)
                if g < 7:
                    prep(g + 1, 1 - b)
                    gstart(1 - b)
                bb16 = jnp.full((16,), b, _I32)

                def edge_body(j, _):
                    j16 = jnp.broadcast_to(j, (16,)).astype(_I32)
                    wj = plsc.load_gather(w_v, [bb16, j16])
                    lj = jnp.max(plsc.load_gather(lidx_v, [bb16, j16]))
                    for k2 in range(DH // 16):
                        sl = pl.ds(k2 * 16, 16)
                        acc_v[lj, sl] = acc_v[lj, sl] + rows_v[b, j, sl] * wj
                    return 0
                lax.fori_loop(0, 16, edge_body, 0)
            return 0
        lax.fori_loop(0, nseg, seg_body, 0)

        def bias_row(j, _):
            for k2 in range(DH // 16):
                sl = pl.ds(k2 * 16, 16)
                acc_v[j, sl] = acc_v[j, sl] + b_v[sl]
            return 0
        lax.fori_loop(0, RNODES, bias_row, 0)
        pltpu.sync_copy(acc_v, out_hbm.at[pl.ds(lo, RNODES)])
        return 0
    lax.fori_loop(0, RPT, range_body, 0)


@functools.cache
def _edge_call():
  return pl.kernel(
    _edge_body,
    out_type=jax.ShapeDtypeStruct((NPAD, DH), _F32),
    mesh=plsc.VectorSubcoreMesh(core_axis_name="c", subcore_axis_name="s",
                                num_cores=2, num_subcores=16),
    compiler_params=pltpu.CompilerParams(needs_layout_passes=False),
    scratch_types=[
        pltpu.VMEM((KR1, 128), _I32),
        pltpu.VMEM((KR1, 128), _I32),
        pltpu.VMEM((NPAD,), _F32),
        pltpu.VMEM((NPAD,), _F32),
        pltpu.VMEM((NPAD,), _F32),
        pltpu.VMEM((144,), _I32),
        pltpu.VMEM((DH,), _F32),
        pltpu.VMEM((2, 128), _F32),
        pltpu.VMEM((RNODES, DH), _F32),
        pltpu.VMEM((2, 16, DH), _F32),
        pltpu.VMEM((2, 16), _I32),
        pltpu.VMEM((2, 16), _I32),
        pltpu.VMEM((2, 16), _F32),
        pltpu.VMEM_SHARED((NPAD,), _F32),
        pltpu.SemaphoreType.DMA,
        pltpu.SemaphoreType.DMA,
        pltpu.SemaphoreType.DMA,
        pltpu.SemaphoreType.DMA,
    ],
  )


def _edge_layer(src3, dst3, srcf, dstf, cuts, as_, ad_, h, b):
    return _edge_call()(src3, dst3, srcf, dstf, cuts, as_, ad_, h, b)


@jax.jit
def kernel(x, edge_index, W1, a_src1, a_dst1, b1, W2, a_src2, a_dst2, b2):
    npad_e = EPAD - E_FULL
    loop = jnp.arange(N, dtype=_I32)
    src = jnp.concatenate([edge_index[0].astype(_I32), loop,
                           jnp.zeros((npad_e,), _I32)])
    dst = jnp.concatenate([edge_index[1].astype(_I32), loop,
                           N + (jnp.arange(npad_e, dtype=_I32) % (NPAD - N))])
    order = jnp.argsort(dst)
    srcf = src[order]
    dstf = dst[order]
    src3 = srcf.reshape(16, KR1, 128)
    dst3 = dstf.reshape(16, KR1, 128)
    cuts = jnp.searchsorted(dstf, jnp.arange(NRANGE, dtype=_I32) * RNODES,
                            side="left").astype(_I32)
    cuts = jnp.concatenate([cuts, jnp.full((144 - NRANGE,), EPAD, _I32)])
    srcf = jnp.concatenate([srcf, jnp.zeros((128,), _I32)])
    dstf = jnp.concatenate([dstf, jnp.full((128,), NPAD - 1, _I32)])

    xp = jnp.pad(x, ((0, NPAD - N), (0, 0)))
    a128_1 = jnp.zeros((DH, 128), _F32).at[:, 0].set(a_src1).at[:, 1].set(a_dst1)
    a128_2 = jnp.zeros((DH, 128), _F32).at[:, 0].set(a_src2).at[:, 1].set(a_dst2)

    h1, av1 = _mm_call(xp, W1, a128_1, relu_in=False)
    out1 = _edge_layer(src3, dst3, srcf, dstf, cuts,
                       av1[:, 0], av1[:, 1], h1, b1)
    h2, av2 = _mm_call(out1, W2, a128_2, relu_in=True)
    out2 = _edge_layer(src3, dst3, srcf, dstf, cuts,
                       av2[:, 0], av2[:, 1], h2, b2)
    return out2[:N]


# speedup vs baseline: 4.5089x; 1.2707x over previous
import functools

import jax
import jax.numpy as jnp
from jax import lax
from jax.experimental import pallas as pl
from jax.experimental.pallas import tpu as pltpu
from jax.experimental.pallas import tpu_sc as plsc

N = 10000
NPAD = 10240
DH = 512
E_RAW = 160000
E_FULL = E_RAW + N
KR1 = 86
NT1 = KR1 * 128
EPAD = NT1 * 16
NRANGE = 128
RNODES = NPAD // NRANGE
RPT = NRANGE // 32
_F32 = jnp.float32
_I32 = jnp.int32


def _mm_call(xp, w, a128, relu_in):
    k = w.shape[0]
    nblk = NPAD // 512

    def body(x_ref, w_ref, a_ref, h_ref, av_ref):
        xb = x_ref[...]
        if relu_in:
            xb = jnp.maximum(xb, 0.0)
        h = jnp.dot(xb, w_ref[...], preferred_element_type=_F32)
        h_ref[...] = h
        av_ref[...] = jnp.dot(h, a_ref[...], preferred_element_type=_F32)

    return pl.pallas_call(
        body,
        grid=(nblk,),
        in_specs=[
            pl.BlockSpec((512, k), lambda i: (i, 0)),
            pl.BlockSpec((k, DH), lambda i: (0, 0)),
            pl.BlockSpec((DH, 128), lambda i: (0, 0)),
        ],
        out_specs=[
            pl.BlockSpec((512, DH), lambda i: (i, 0)),
            pl.BlockSpec((512, 128), lambda i: (i, 0)),
        ],
        out_shape=[
            jax.ShapeDtypeStruct((NPAD, DH), _F32),
            jax.ShapeDtypeStruct((NPAD, 128), _F32),
        ],
    )(xp, w, a128)


def _edge_body(src3_hbm, dst3_hbm, srcf_hbm, dstf_hbm, cuts_hbm,
               as_hbm, ad_hbm, h_hbm, b_hbm, out_hbm,
               src_v, dst_v, as_v, ad_v, dn_v, cuts_v, b_v, est_v,
               acc_v, rows_v, idx_v, lidx_v, w_v, denomS,
               dsem0, dsem1, gsem0, gsem1):
    c = lax.axis_index("c")
    s = lax.axis_index("s")
    dsems = (dsem0, dsem1)
    gsems = (gsem0, gsem1)

    pltpu.sync_copy(src3_hbm.at[s], src_v)
    pltpu.sync_copy(dst3_hbm.at[s], dst_v)
    pltpu.sync_copy(as_hbm, as_v)
    pltpu.sync_copy(ad_hbm, ad_v)
    pltpu.sync_copy(cuts_hbm, cuts_v)
    pltpu.sync_copy(b_hbm, b_v)

    for q in range(8):
        est_v[0, pl.ds(q * 16, 16)] = jnp.zeros((16,), _F32)
    for q in range(5):
        pltpu.sync_copy(est_v.at[0], denomS.at[pl.ds(s * 640 + q * 128, 128)])
    plsc.subcore_barrier()

    def eexp_row(kk, b):
        def g_body(g, _):
            sl = pl.ds(g * 16, 16)
            s16 = src_v[kk, sl]
            d16 = dst_v[kk, sl]
            av = plsc.load_gather(as_v, [s16])
            dv = plsc.load_gather(ad_v, [d16])
            e = av + dv
            e = jnp.where(e >= 0.0, e, 0.2 * e)
            est_v[b, sl] = jnp.exp(e)
            return 0
        lax.fori_loop(0, 8, g_body, 0)

    def p1_pair(r2, _):
        for b in range(2):
            r = r2 * 2 + b

            @pl.when(r >= 2)
            def _():
                pltpu.make_async_copy(est_v.at[b], denomS.at[dst_v.at[r - 2]],
                                      dsems[b]).wait()
            eexp_row(r, b)
            pltpu.async_copy(est_v.at[b], denomS.at[dst_v.at[r]], dsems[b],
                             add=True)
        return 0
    lax.fori_loop(0, KR1 // 2, p1_pair, 0)
    pltpu.make_async_copy(est_v.at[0], denomS.at[dst_v.at[KR1 - 2]],
                          dsems[0]).wait()
    pltpu.make_async_copy(est_v.at[1], denomS.at[dst_v.at[KR1 - 1]],
                          dsems[1]).wait()
    plsc.subcore_barrier()

    pltpu.sync_copy(denomS, dn_v)

    w_id = c * 16 + s
    lanes = lax.iota(_I32, 16)

    def scalar_at(r):
        v = plsc.load_gather(cuts_v, [jnp.broadcast_to(r, (16,)).astype(_I32)])
        return jnp.max(v)

    def range_body(rp, _):
        rid = w_id * RPT + rp
        lo = pl.multiple_of(rid * RNODES, RNODES)
        hi = lo + RNODES
        e_start = scalar_at(rid)
        e_end = scalar_at(rid + 1)
        e0 = jnp.bitwise_and(e_start, jnp.int32(-16))
        nseg = (e_end - e0 + 127) // 128

        def zacc(j, _):
            for k2 in range(DH // 16):
                acc_v[j, pl.ds(k2 * 16, 16)] = jnp.zeros((16,), _F32)
            return 0
        lax.fori_loop(0, RNODES, zacc, 0)

        def prep(g, bb):
            sl = pl.ds(g * 16, 16)
            s16 = src_v[0, sl]
            d16 = dst_v[0, sl]
            av = plsc.load_gather(as_v, [s16])
            dv = plsc.load_gather(ad_v, [d16])
            e = av + dv
            e = jnp.where(e >= 0.0, e, 0.2 * e)
            dn = plsc.load_gather(dn_v, [d16])
            alpha = jnp.exp(e) / (dn + 1e-16)
            inr = (d16 >= lo) & (d16 < hi)
            idx_v[bb, :] = s16
            lidx_v[bb, :] = jnp.where(inr, d16 - lo, 0)
            w_v[bb, :] = jnp.where(inr, alpha, 0.0)

        def gstart(bb):
            pltpu.async_copy(h_hbm.at[idx_v.at[bb]], rows_v.at[bb], gsems[bb])

        def gwait(bb):
            pltpu.make_async_copy(h_hbm.at[idx_v.at[bb]], rows_v.at[bb],
                                  gsems[bb]).wait()

        def seg_body(si, _):
            eb = pl.multiple_of(e0 + si * 128, 16)
            pltpu.sync_copy(srcf_hbm.at[pl.ds(eb, 128)], src_v.at[0])
            pltpu.sync_copy(dstf_hbm.at[pl.ds(eb, 128)], dst_v.at[0])
            prep(0, 0)
            gstart(0)
            for g in range(8):
                b = g % 2
                gwait(b)
                if g < 7:
                    prep(g + 1, 1 - b)
                    gstart(1 - b)
                bb16 = jnp.full((16,), b, _I32)

                def edge_body(j, _):
                    j16 = jnp.broadcast_to(j, (16,)).astype(_I32)
                    wj = plsc.load_gather(w_v, [bb16, j16])
                    lj = jnp.max(plsc.load_gather(lidx_v, [bb16, j16]))
                    for k2 in range(DH // 16):
                        sl = pl.ds(k2 * 16, 16)
                        acc_v[lj, sl] = acc_v[lj, sl] + rows_v[b, j, sl] * wj
                    return 0
                lax.fori_loop(0, 16, edge_body, 0)
            return 0
        lax.fori_loop(0, nseg, seg_body, 0)

        def bias_row(j, _):
            for k2 in range(DH // 16):
                sl = pl.ds(k2 * 16, 16)
                acc_v[j, sl] = acc_v[j, sl] + b_v[sl]
            return 0
        lax.fori_loop(0, RNODES, bias_row, 0)
        pltpu.sync_copy(acc_v, out_hbm.at[pl.ds(lo, RNODES)])
        return 0
    lax.fori_loop(0, RPT, range_body, 0)


@functools.cache
def _edge_call():
  return pl.kernel(
    _edge_body,
    out_type=jax.ShapeDtypeStruct((NPAD, DH), _F32),
    mesh=plsc.VectorSubcoreMesh(core_axis_name="c", subcore_axis_name="s",
                                num_cores=2, num_subcores=16),
    compiler_params=pltpu.CompilerParams(needs_layout_passes=False),
    scratch_types=[
        pltpu.VMEM((KR1, 128), _I32),
        pltpu.VMEM((KR1, 128), _I32),
        pltpu.VMEM((NPAD,), _F32),
        pltpu.VMEM((NPAD,), _F32),
        pltpu.VMEM((NPAD,), _F32),
        pltpu.VMEM((144,), _I32),
        pltpu.VMEM((DH,), _F32),
        pltpu.VMEM((2, 128), _F32),
        pltpu.VMEM((RNODES, DH), _F32),
        pltpu.VMEM((2, 16, DH), _F32),
        pltpu.VMEM((2, 16), _I32),
        pltpu.VMEM((2, 16), _I32),
        pltpu.VMEM((2, 16), _F32),
        pltpu.VMEM_SHARED((NPAD,), _F32),
        pltpu.SemaphoreType.DMA,
        pltpu.SemaphoreType.DMA,
        pltpu.SemaphoreType.DMA,
        pltpu.SemaphoreType.DMA,
    ],
  )


def _edge_layer(src3, dst3, srcf, dstf, cuts, as_, ad_, h, b):
    return _edge_call()(src3, dst3, srcf, dstf, cuts, as_, ad_, h, b)


@jax.jit
def kernel(x, edge_index, W1, a_src1, a_dst1, b1, W2, a_src2, a_dst2, b2):
    npad_e = EPAD - E_FULL
    loop = jnp.arange(N, dtype=_I32)
    src = jnp.concatenate([edge_index[0].astype(_I32), loop,
                           jnp.zeros((npad_e,), _I32)])
    dst = jnp.concatenate([edge_index[1].astype(_I32), loop,
                           N + (jnp.arange(npad_e, dtype=_I32) % (NPAD - N))])
    order = jnp.argsort(dst)
    srcf = src[order]
    dstf = dst[order]
    src3 = srcf.reshape(16, KR1, 128)
    dst3 = dstf.reshape(16, KR1, 128)
    cuts = jnp.searchsorted(dstf, jnp.arange(NRANGE, dtype=_I32) * RNODES,
                            side="left").astype(_I32)
    cuts = jnp.minimum(cuts, E_FULL)
    cuts = jnp.concatenate([cuts, jnp.full((144 - NRANGE,), E_FULL, _I32)])
    srcf = jnp.concatenate([srcf, jnp.zeros((128,), _I32)])
    dstf = jnp.concatenate([dstf, jnp.full((128,), NPAD - 1, _I32)])

    xp = jnp.pad(x, ((0, NPAD - N), (0, 0)))
    a128_1 = jnp.zeros((DH, 128), _F32).at[:, 0].set(a_src1).at[:, 1].set(a_dst1)
    a128_2 = jnp.zeros((DH, 128), _F32).at[:, 0].set(a_src2).at[:, 1].set(a_dst2)

    h1, av1 = _mm_call(xp, W1, a128_1, relu_in=False)
    out1 = _edge_layer(src3, dst3, srcf, dstf, cuts,
                       av1[:, 0], av1[:, 1], h1, b1)
    h2, av2 = _mm_call(out1, W2, a128_2, relu_in=True)
    out2 = _edge_layer(src3, dst3, srcf, dstf, cuts,
                       av2[:, 0], av2[:, 1], h2, b2)
    return out2[:N]
